# baseline (speedup 1.0000x reference)
import jax
import jax.numpy as jnp
from jax import lax
from jax.experimental import pallas as pl
from jax.experimental.pallas import tpu as pltpu
from jax.experimental.pallas import tpu_sc as plsc

_ALT_VALUES = (150, 200, 250, 300)
_NUM_PARTS = 16
_NUM_ROWS = 5
_BATCH = 16384
_NC, _NS = 2, 16
_NW = _NC * _NS
_BPW = _BATCH // _NW
_GROUPS = _BPW // 16


def _sc_kernel(att_hbm, alt_hbm, temp_hbm, out_hbm, att_v, temp_v, alt_v,
               table_v, rows_v):
    wid = lax.axis_index("s") * _NC + lax.axis_index("c")
    base = wid * _BPW
    pltpu.sync_copy(att_hbm, att_v)
    pltpu.sync_copy(temp_hbm, temp_v)
    pltpu.sync_copy(alt_hbm.at[pl.ds(base, _BPW)], alt_v)

    recip = 1.0 / jnp.maximum(jnp.abs(temp_v[...]), jnp.float32(0.1))
    for i in range(_NUM_ROWS):
        w = att_v[i] * recip
        e = jnp.exp(w - jnp.max(w))
        table_v[pl.ds(i * _NUM_PARTS, _NUM_PARTS)] = e / jnp.sum(e)

    lane = lax.iota(jnp.int32, 16)
    for g in range(_GROUPS):
        a = alt_v[pl.ds(g * 16, 16)]
        idx = jnp.full((16,), 4, dtype=jnp.int32)
        for i, v in enumerate(_ALT_VALUES):
            idx = jnp.where(a == jnp.int32(v), jnp.int32(i), idx)
        src = idx * _NUM_PARTS
        rid = g * 16 + lane
        for l in range(_NUM_PARTS):
            col = plsc.load_gather(table_v, [src + l])
            plsc.store_scatter(rows_v, [rid, jnp.full((16,), l, jnp.int32)],
                               col)

    pltpu.sync_copy(rows_v, out_hbm.at[pl.ds(base, _BPW)])


def kernel(altitudes, attention, temp):
    mesh = plsc.VectorSubcoreMesh(core_axis_name="c", subcore_axis_name="s")
    run = pl.kernel(
        _sc_kernel,
        out_type=jax.ShapeDtypeStruct((_BATCH, _NUM_PARTS), jnp.float32),
        mesh=mesh,
        compiler_params=pltpu.CompilerParams(
            use_tc_tiling_on_sc=False, needs_layout_passes=False),
        scratch_types=[
            pltpu.VMEM((_NUM_ROWS, _NUM_PARTS), jnp.float32),
            pltpu.VMEM((16,), jnp.float32),
            pltpu.VMEM((_BPW,), jnp.int32),
            pltpu.VMEM((_NUM_ROWS * _NUM_PARTS,), jnp.float32),
            pltpu.VMEM((_BPW, _NUM_PARTS), jnp.float32),
        ],
    )
    temp16 = jnp.broadcast_to(jnp.asarray(temp, jnp.float32).reshape(1), (16,))
    return run(attention, altitudes, temp16)


# speedup vs baseline: 1.0112x; 1.0112x over previous
import jax
import jax.numpy as jnp
from jax import lax
from jax.experimental import pallas as pl
from jax.experimental.pallas import tpu as pltpu
from jax.experimental.pallas import tpu_sc as plsc

_ALT_VALUES = (150, 200, 250, 300)
_NUM_PARTS = 16
_NUM_ROWS = 5
_BATCH = 16384
_NC, _NS = 2, 16
_NW = _NC * _NS
_BPW = _BATCH // _NW
_GROUPS = _BPW // 16


def _sc_kernel(att_hbm, alt_hbm, temp_hbm, out_hbm, att_v, temp_v, alt_v,
               table_v, rows_v):
    wid = lax.axis_index("s") * _NC + lax.axis_index("c")
    base = wid * _BPW
    pltpu.sync_copy(att_hbm, att_v)
    pltpu.sync_copy(temp_hbm, temp_v)
    pltpu.sync_copy(alt_hbm.at[pl.ds(base, _BPW)], alt_v)

    recip = 1.0 / jnp.maximum(jnp.abs(temp_v[...]), jnp.float32(0.1))
    for i in range(_NUM_ROWS):
        w = att_v[i] * recip
        e = jnp.exp(w - jnp.max(w))
        table_v[pl.ds(i * _NUM_PARTS, _NUM_PARTS)] = e / jnp.sum(e)

    lane = lax.iota(jnp.int32, 16)
    for g in range(_GROUPS):
        a = alt_v[pl.ds(g * 16, 16)]
        idx = jnp.full((16,), 4, dtype=jnp.int32)
        for i, v in enumerate(_ALT_VALUES):
            idx = jnp.where(a == jnp.int32(v), jnp.int32(i), idx)
        src = idx * _NUM_PARTS
        rid = g * 16 + lane
        for l in range(_NUM_PARTS):
            col = plsc.load_gather(table_v, [src + l])
            plsc.store_scatter(rows_v, [rid, jnp.full((16,), l, jnp.int32)],
                               col)

    pltpu.sync_copy(rows_v, out_hbm.at[pl.ds(base, _BPW)])


def kernel(altitudes, attention, temp):
    mesh = plsc.VectorSubcoreMesh(core_axis_name="c", subcore_axis_name="s")
    run = pl.kernel(
        _sc_kernel,
        out_type=jax.ShapeDtypeStruct((_BATCH, _NUM_PARTS), jnp.float32),
        mesh=mesh,
        compiler_params=pltpu.CompilerParams(
            use_tc_tiling_on_sc=True, needs_layout_passes=False),
        scratch_types=[
            pltpu.VMEM((_NUM_ROWS, _NUM_PARTS), jnp.float32),
            pltpu.VMEM((16,), jnp.float32),
            pltpu.VMEM((_BPW,), jnp.int32),
            pltpu.VMEM((_NUM_ROWS * _NUM_PARTS,), jnp.float32),
            pltpu.VMEM((_BPW, _NUM_PARTS), jnp.float32),
        ],
    )
    temp16 = jnp.broadcast_to(jnp.asarray(temp, jnp.float32).reshape(1), (16,))
    return run(attention, altitudes, temp16)
